# baseline (speedup 1.0000x reference)
import functools
import jax
import jax.numpy as jnp
from jax import lax
from jax.experimental import pallas as pl
from jax.experimental.pallas import tpu as pltpu
from jax.experimental.pallas import tpu_sc as plsc

NN = 10000
EE = 320000
DD = 128
NG = 64
NCLS = 10

NC = 2
NS = 16
NW = NC * NS
KCH = 128
NCHUNK = 80
EPT = KCH * NCHUNK
EPAD = EPT * NW
NP = 10240
RPT = NP // NS

_mesh = plsc.VectorSubcoreMesh(core_axis_name="c", subcore_axis_name="s")


def _zero_rows(ref, nrows, ncolchunks, val=0.0):
    v = jnp.full((16,), val, jnp.float32)

    def body(r, _):
        for k in range(ncolchunks):
            ref[r, pl.ds(k * 16, 16)] = v
        return 0

    lax.fori_loop(0, nrows, body, 0)


@functools.partial(
    pl.kernel,
    out_type=jax.ShapeDtypeStruct((NC, NP, DD), jnp.float32),
    mesh=_mesh,
    scratch_types=[
        pltpu.VMEM((NCHUNK, KCH), jnp.int32),
        pltpu.VMEM((KCH, DD), jnp.float32),
        pltpu.SemaphoreType.DMA,
        pltpu.VMEM_SHARED((NP, DD), jnp.float32),
    ],
)
def _deg_sc(didx_hbm, degp_hbm, didx_v, ones_v, sem, acc_sh):
    cid = lax.axis_index("c")
    sid = lax.axis_index("s")
    wid = sid * NC + cid

    _zero_rows(ones_v, KCH, DD // 16, 0.0)

    def zbody(c, _):
        pltpu.sync_copy(ones_v, acc_sh.at[pl.ds(sid * RPT + c * KCH, KCH)])
        return 0

    lax.fori_loop(0, RPT // KCH, zbody, 0)
    plsc.subcore_barrier()

    _zero_rows(ones_v, KCH, DD // 16, 1.0)
    pltpu.sync_copy(didx_hbm.at[wid], didx_v)

    WAVE = 4

    def body(j, _):
        for b in range(WAVE):
            pltpu.async_copy(ones_v, acc_sh.at[didx_v.at[j * WAVE + b]], sem,
                             add=True)
        for b in range(WAVE):
            pltpu.make_async_copy(degp_hbm.at[cid, pl.ds(0, KCH)], ones_v,
                                  sem).wait()
        return 0

    lax.fori_loop(0, NCHUNK // WAVE, body, 0)
    plsc.subcore_barrier()
    pltpu.sync_copy(acc_sh.at[pl.ds(sid * RPT, RPT)],
                    degp_hbm.at[cid, pl.ds(sid * RPT, RPT)])


@functools.partial(
    pl.kernel,
    out_type=jax.ShapeDtypeStruct((NC, NP, DD), jnp.float32),
    mesh=_mesh,
    scratch_types=[
        pltpu.VMEM((2 * NCHUNK, KCH), jnp.int32),
        pltpu.VMEM((KCH, DD), jnp.float32),
        pltpu.SemaphoreType.DMA,
        pltpu.VMEM_SHARED((NP, DD), jnp.float32),
    ],
)
def _edge_sc(y_hbm, eidx_hbm, accp_hbm, eidx_v, rows_v, sem, acc_sh):
    cid = lax.axis_index("c")
    sid = lax.axis_index("s")
    wid = sid * NC + cid

    _zero_rows(rows_v, KCH, DD // 16, 0.0)

    def zbody(c, _):
        pltpu.sync_copy(rows_v, acc_sh.at[pl.ds(sid * RPT + c * KCH, KCH)])
        return 0

    lax.fori_loop(0, RPT // KCH, zbody, 0)
    plsc.subcore_barrier()

    pltpu.sync_copy(eidx_hbm.at[wid], eidx_v)

    def body(c, _):
        pltpu.async_copy(y_hbm.at[eidx_v.at[c]], rows_v, sem).wait()
        pltpu.sync_copy(rows_v, acc_sh.at[eidx_v.at[NCHUNK + c]], add=True)
        return 0

    lax.fori_loop(0, NCHUNK, body, 0)
    plsc.subcore_barrier()
    pltpu.sync_copy(acc_sh.at[pl.ds(sid * RPT, RPT)],
                    accp_hbm.at[cid, pl.ds(sid * RPT, RPT)])


def _dot(a, b):
    return lax.dot_general(a, b, (((1,), (0,)), ((), ())),
                           precision=lax.Precision.HIGHEST,
                           preferred_element_type=jnp.float32)


def _dinv(d0, d1):
    return lax.rsqrt(d0 + d1 + 1.0)


BLK = 1024
NBLK = NP // BLK


def _p1_body(x_ref, d0_ref, d1_ref, w_ref, y_ref):
    dinv = _dinv(d0_ref[...], d1_ref[...])
    y_ref[...] = _dot(x_ref[...], w_ref[...]) * dinv


def _p1(x_pad, d0, d1, W1):
    return pl.pallas_call(
        _p1_body,
        grid=(NBLK,),
        in_specs=[
            pl.BlockSpec((BLK, DD), lambda i: (i, 0)),
            pl.BlockSpec((BLK, 1), lambda i: (i, 0)),
            pl.BlockSpec((BLK, 1), lambda i: (i, 0)),
            pl.BlockSpec((DD, DD), lambda i: (0, 0)),
        ],
        out_specs=pl.BlockSpec((BLK, DD), lambda i: (i, 0)),
        out_shape=jax.ShapeDtypeStruct((NP, DD), jnp.float32),
    )(x_pad, d0, d1, W1)


def _p3_body(a0_ref, a1_ref, y_ref, d0_ref, d1_ref, b_ref, w_ref, out_ref):
    dinv = _dinv(d0_ref[...], d1_ref[...])
    h = jax.nn.relu(dinv * (a0_ref[...] + a1_ref[...] + y_ref[...]) + b_ref[...])
    out_ref[...] = _dot(h, w_ref[...]) * dinv


def _p3(a0, a1, y, d0, d1, b, Wn):
    return pl.pallas_call(
        _p3_body,
        grid=(NBLK,),
        in_specs=[
            pl.BlockSpec((BLK, DD), lambda i: (i, 0)),
            pl.BlockSpec((BLK, DD), lambda i: (i, 0)),
            pl.BlockSpec((BLK, DD), lambda i: (i, 0)),
            pl.BlockSpec((BLK, 1), lambda i: (i, 0)),
            pl.BlockSpec((BLK, 1), lambda i: (i, 0)),
            pl.BlockSpec((1, DD), lambda i: (0, 0)),
            pl.BlockSpec((DD, DD), lambda i: (0, 0)),
        ],
        out_specs=pl.BlockSpec((BLK, DD), lambda i: (i, 0)),
        out_shape=jax.ShapeDtypeStruct((NP, DD), jnp.float32),
    )(a0, a1, y, d0, d1, b, Wn)


def _p7_body(a0_ref, a1_ref, y_ref, d0_ref, d1_ref, b_ref, g_ref, be_ref,
             h_ref, ss_ref, acc_ref):
    i = pl.program_id(0)
    dinv = _dinv(d0_ref[...], d1_ref[...])
    h = jax.nn.relu(dinv * (a0_ref[...] + a1_ref[...] + y_ref[...]) + b_ref[...])
    h_ref[...] = h

    rid = i * BLK + lax.broadcasted_iota(jnp.int32, (BLK, 1), 0)
    hm = jnp.where(rid < NN, h, 0.0)

    @pl.when(i == 0)
    def _():
        acc_ref[...] = jnp.zeros_like(acc_ref)

    acc_ref[0:1, :] += jnp.sum(hm, axis=0, keepdims=True)
    acc_ref[1:2, :] += jnp.sum(hm * hm, axis=0, keepdims=True)

    @pl.when(i == NBLK - 1)
    def _():
        mean = acc_ref[0:1, :] / NN
        var = acc_ref[1:2, :] / NN - mean * mean
        scale = g_ref[...] / jnp.sqrt(var + 1e-5)
        shift = be_ref[...] - mean * scale
        ss_ref[...] = jnp.concatenate(
            [scale, shift, jnp.zeros((6, DD), jnp.float32)], axis=0)


def _p7(a0, a1, y, d0, d1, b, gamma, beta):
    return pl.pallas_call(
        _p7_body,
        grid=(NBLK,),
        in_specs=[
            pl.BlockSpec((BLK, DD), lambda i: (i, 0)),
            pl.BlockSpec((BLK, DD), lambda i: (i, 0)),
            pl.BlockSpec((BLK, DD), lambda i: (i, 0)),
            pl.BlockSpec((BLK, 1), lambda i: (i, 0)),
            pl.BlockSpec((BLK, 1), lambda i: (i, 0)),
            pl.BlockSpec((1, DD), lambda i: (0, 0)),
            pl.BlockSpec((1, DD), lambda i: (0, 0)),
            pl.BlockSpec((1, DD), lambda i: (0, 0)),
        ],
        out_specs=[
            pl.BlockSpec((BLK, DD), lambda i: (i, 0)),
            pl.BlockSpec((8, DD), lambda i: (0, 0)),
        ],
        out_shape=[
            jax.ShapeDtypeStruct((NP, DD), jnp.float32),
            jax.ShapeDtypeStruct((8, DD), jnp.float32),
        ],
        scratch_shapes=[pltpu.VMEM((8, DD), jnp.float32)],
    )(a0, a1, y, d0, d1, b, gamma, beta)


def _p8_body(h_ref, bat_ref, ss_ref, out_ref):
    i = pl.program_id(0)

    @pl.when(i == 0)
    def _():
        out_ref[...] = jnp.full((NG, DD), -jnp.inf, jnp.float32)

    hn = h_ref[...] * ss_ref[0:1, :] + ss_ref[1:2, :]
    bat = bat_ref[...]

    def body(g, _):
        mask = bat == g
        mm = jnp.max(jnp.where(mask, hn, -jnp.inf), axis=0, keepdims=True)
        cur = out_ref[pl.ds(g, 1), :]
        out_ref[pl.ds(g, 1), :] = jnp.maximum(cur, mm)
        return 0

    lax.fori_loop(0, NG, body, 0)


def _p8(h3, batp, ss):
    return pl.pallas_call(
        _p8_body,
        grid=(NBLK,),
        in_specs=[
            pl.BlockSpec((BLK, DD), lambda i: (i, 0)),
            pl.BlockSpec((BLK, 1), lambda i: (i, 0)),
            pl.BlockSpec((8, DD), lambda i: (0, 0)),
        ],
        out_specs=pl.BlockSpec((NG, DD), lambda i: (0, 0)),
        out_shape=jax.ShapeDtypeStruct((NG, DD), jnp.float32),
    )(h3, batp, ss)


def _p9_body(g_ref, w1_ref, b1_ref, w2_ref, b2_ref, w3_ref, b3_ref, out_ref):
    g = g_ref[...]
    g = jnp.where(g == -jnp.inf, 0.0, g)
    g = jax.nn.relu(_dot(g, w1_ref[...]) + b1_ref[...])
    g = jax.nn.relu(_dot(g, w2_ref[...]) + b2_ref[...])
    out_ref[...] = _dot(g, w3_ref[...]) + b3_ref[...]


def _p9(gmax, lW1, lb1, lW2p, lb2p, lW3p, lb3p):
    return pl.pallas_call(
        _p9_body,
        out_shape=jax.ShapeDtypeStruct((NG, DD), jnp.float32),
    )(gmax, lW1, lb1, lW2p, lb2p, lW3p, lb3p)


def kernel(x, edge_index, batch, W1, b1, W2, b2, W3, b3, gamma, beta,
           lW1, lb1, lW2, lb2, lW3, lb3):
    src = edge_index[0]
    dst = edge_index[1]
    npad = EPAD - EE
    sidx3 = jnp.concatenate([src, jnp.zeros((npad,), jnp.int32)]).reshape(
        NW, NCHUNK, KCH)
    didx3 = jnp.concatenate([dst, jnp.full((npad,), NP - 1, jnp.int32)]).reshape(
        NW, NCHUNK, KCH)
    eidx3 = jnp.concatenate([sidx3, didx3], axis=1)
    x_pad = jnp.concatenate([x, jnp.zeros((NP - NN, DD), jnp.float32)], axis=0)
    batp = jnp.concatenate([batch, jnp.full((NP - NN,), NG, jnp.int32)]).reshape(
        NP, 1)

    b1r = b1.reshape(1, DD)
    b2r = b2.reshape(1, DD)
    b3r = b3.reshape(1, DD)
    gr = gamma.reshape(1, DD)
    ber = beta.reshape(1, DD)
    lb1r = lb1.reshape(1, DD)
    lW2p = jnp.pad(lW2, ((0, 0), (0, DD - lW2.shape[1])))
    lb2p = jnp.pad(lb2, (0, DD - lb2.shape[0])).reshape(1, DD)
    lW3p = jnp.pad(lW3, ((0, DD - lW3.shape[0]), (0, DD - lW3.shape[1])))
    lb3p = jnp.pad(lb3, (0, DD - lb3.shape[0])).reshape(1, DD)

    degp = _deg_sc(didx3)
    d0 = lax.slice(degp[0], (0, 0), (NP, 1))
    d1 = lax.slice(degp[1], (0, 0), (NP, 1))

    y1 = _p1(x_pad, d0, d1, W1)
    a1 = _edge_sc(y1, eidx3)
    y2 = _p3(a1[0], a1[1], y1, d0, d1, b1r, W2)
    a2 = _edge_sc(y2, eidx3)
    y3 = _p3(a2[0], a2[1], y2, d0, d1, b2r, W3)
    a3 = _edge_sc(y3, eidx3)
    h3, ss = _p7(a3[0], a3[1], y3, d0, d1, b3r, gr, ber)
    gmax = _p8(h3, batp, ss)
    out = _p9(gmax, lW1, lb1r, lW2p, lb2p, lW3p, lb3p)
    return out[:, :NCLS]


# speedup vs baseline: 1.2680x; 1.2680x over previous
import functools
import jax
import jax.numpy as jnp
from jax import lax
from jax.experimental import pallas as pl
from jax.experimental.pallas import tpu as pltpu
from jax.experimental.pallas import tpu_sc as plsc

NN = 10000
EE = 320000
DD = 128
NG = 64
NCLS = 10

NC = 2
NS = 16
NW = NC * NS
KCH = 128
CPS = 160
TOTCH = NS * CPS
EPAD = TOTCH * KCH
CA = 120
CB = CPS - CA
CDG = 80
CAM = max(CA, CB)
NP = 10240
RPT = NP // NS

_mesh = plsc.VectorSubcoreMesh(core_axis_name="c", subcore_axis_name="s")


def _zero_rows(ref, nrows, ncolchunks, val=0.0):
    v = jnp.full((16,), val, jnp.float32)

    def body(r, _):
        for k in range(ncolchunks):
            ref[r, pl.ds(k * 16, 16)] = v
        return 0

    lax.fori_loop(0, nrows, body, 0)


@functools.partial(
    pl.kernel,
    out_type=jax.ShapeDtypeStruct((NC, NP, DD), jnp.float32),
    mesh=_mesh,
    scratch_types=[
        pltpu.VMEM((CPS, KCH), jnp.int32),
        pltpu.VMEM((KCH, DD), jnp.float32),
        pltpu.VMEM_SHARED((NP, DD), jnp.float32),
    ],
)
def _deg_sc(eidx_hbm, degp_hbm, didx_v, ones_v, acc_sh):
    cid = lax.axis_index("c")
    sid = lax.axis_index("s")

    _zero_rows(ones_v, KCH, DD // 16, 0.0)

    def zbody(c, _):
        pltpu.sync_copy(ones_v, acc_sh.at[pl.ds(sid * RPT + c * KCH, KCH)])
        return 0

    lax.fori_loop(0, RPT // KCH, zbody, 0)
    plsc.subcore_barrier()

    _zero_rows(ones_v, KCH, DD // 16, 1.0)
    off = cid * CDG
    n = jnp.where(cid == 0, CDG, CPS - CDG)
    pltpu.sync_copy(eidx_hbm.at[sid, pl.ds(CPS, CPS)], didx_v)

    def body(c, _):
        pltpu.sync_copy(ones_v, acc_sh.at[didx_v.at[off + c]], add=True)
        return 0

    lax.fori_loop(0, n, body, 0)
    plsc.subcore_barrier()
    pltpu.sync_copy(acc_sh.at[pl.ds(sid * RPT, RPT)],
                    degp_hbm.at[cid, pl.ds(sid * RPT, RPT)])


@functools.partial(
    pl.kernel,
    out_type=jax.ShapeDtypeStruct((NC, NP, DD), jnp.float32),
    mesh=_mesh,
    scratch_types=[
        pltpu.VMEM((2 * CAM, KCH), jnp.int32),
        pltpu.VMEM((KCH, DD), jnp.float32),
        pltpu.SemaphoreType.DMA,
        pltpu.VMEM_SHARED((NP, DD), jnp.float32),
    ],
)
def _edge_sc(y_hbm, eidx_hbm, accp_hbm, eidx_v, rows_v, sem, acc_sh):
    cid = lax.axis_index("c")
    sid = lax.axis_index("s")

    _zero_rows(rows_v, KCH, DD // 16, 0.0)

    def zbody(c, _):
        pltpu.sync_copy(rows_v, acc_sh.at[pl.ds(sid * RPT + c * KCH, KCH)])
        return 0

    lax.fori_loop(0, RPT // KCH, zbody, 0)
    plsc.subcore_barrier()

    wid = sid * NC + cid
    n = jnp.where(cid == 0, CA, CB)
    pltpu.sync_copy(eidx_hbm.at[wid], eidx_v)

    def body(c, _):
        pltpu.async_copy(y_hbm.at[eidx_v.at[c]], rows_v, sem).wait()
        pltpu.sync_copy(rows_v, acc_sh.at[eidx_v.at[CAM + c]], add=True)
        return 0

    lax.fori_loop(0, n, body, 0)
    plsc.subcore_barrier()
    pltpu.sync_copy(acc_sh.at[pl.ds(sid * RPT, RPT)],
                    accp_hbm.at[cid, pl.ds(sid * RPT, RPT)])


def _dot(a, b):
    return lax.dot_general(a, b, (((1,), (0,)), ((), ())),
                           precision=lax.Precision.HIGHEST,
                           preferred_element_type=jnp.float32)


def _dinv(d0, d1):
    return lax.rsqrt(d0 + d1 + 1.0)


BLK = 1024
NBLK = NP // BLK


def _p1_body(x_ref, d0_ref, d1_ref, w_ref, y_ref):
    dinv = _dinv(d0_ref[...], d1_ref[...])
    y_ref[...] = _dot(x_ref[...], w_ref[...]) * dinv


def _p1(x_pad, d0, d1, W1):
    return pl.pallas_call(
        _p1_body,
        grid=(NBLK,),
        in_specs=[
            pl.BlockSpec((BLK, DD), lambda i: (i, 0)),
            pl.BlockSpec((BLK, 1), lambda i: (i, 0)),
            pl.BlockSpec((BLK, 1), lambda i: (i, 0)),
            pl.BlockSpec((DD, DD), lambda i: (0, 0)),
        ],
        out_specs=pl.BlockSpec((BLK, DD), lambda i: (i, 0)),
        out_shape=jax.ShapeDtypeStruct((NP, DD), jnp.float32),
    )(x_pad, d0, d1, W1)


def _p3_body(a0_ref, a1_ref, y_ref, d0_ref, d1_ref, b_ref, w_ref, out_ref):
    dinv = _dinv(d0_ref[...], d1_ref[...])
    h = jax.nn.relu(dinv * (a0_ref[...] + a1_ref[...] + y_ref[...]) + b_ref[...])
    out_ref[...] = _dot(h, w_ref[...]) * dinv


def _p3(a0, a1, y, d0, d1, b, Wn):
    return pl.pallas_call(
        _p3_body,
        grid=(NBLK,),
        in_specs=[
            pl.BlockSpec((BLK, DD), lambda i: (i, 0)),
            pl.BlockSpec((BLK, DD), lambda i: (i, 0)),
            pl.BlockSpec((BLK, DD), lambda i: (i, 0)),
            pl.BlockSpec((BLK, 1), lambda i: (i, 0)),
            pl.BlockSpec((BLK, 1), lambda i: (i, 0)),
            pl.BlockSpec((1, DD), lambda i: (0, 0)),
            pl.BlockSpec((DD, DD), lambda i: (0, 0)),
        ],
        out_specs=pl.BlockSpec((BLK, DD), lambda i: (i, 0)),
        out_shape=jax.ShapeDtypeStruct((NP, DD), jnp.float32),
    )(a0, a1, y, d0, d1, b, Wn)


def _p7_body(a0_ref, a1_ref, y_ref, d0_ref, d1_ref, b_ref, g_ref, be_ref,
             h_ref, ss_ref, acc_ref):
    i = pl.program_id(0)
    dinv = _dinv(d0_ref[...], d1_ref[...])
    h = jax.nn.relu(dinv * (a0_ref[...] + a1_ref[...] + y_ref[...]) + b_ref[...])
    h_ref[...] = h

    rid = i * BLK + lax.broadcasted_iota(jnp.int32, (BLK, 1), 0)
    hm = jnp.where(rid < NN, h, 0.0)

    @pl.when(i == 0)
    def _():
        acc_ref[...] = jnp.zeros_like(acc_ref)

    acc_ref[0:1, :] += jnp.sum(hm, axis=0, keepdims=True)
    acc_ref[1:2, :] += jnp.sum(hm * hm, axis=0, keepdims=True)

    @pl.when(i == NBLK - 1)
    def _():
        mean = acc_ref[0:1, :] / NN
        var = acc_ref[1:2, :] / NN - mean * mean
        scale = g_ref[...] / jnp.sqrt(var + 1e-5)
        shift = be_ref[...] - mean * scale
        ss_ref[...] = jnp.concatenate(
            [scale, shift, jnp.zeros((6, DD), jnp.float32)], axis=0)


def _p7(a0, a1, y, d0, d1, b, gamma, beta):
    return pl.pallas_call(
        _p7_body,
        grid=(NBLK,),
        in_specs=[
            pl.BlockSpec((BLK, DD), lambda i: (i, 0)),
            pl.BlockSpec((BLK, DD), lambda i: (i, 0)),
            pl.BlockSpec((BLK, DD), lambda i: (i, 0)),
            pl.BlockSpec((BLK, 1), lambda i: (i, 0)),
            pl.BlockSpec((BLK, 1), lambda i: (i, 0)),
            pl.BlockSpec((1, DD), lambda i: (0, 0)),
            pl.BlockSpec((1, DD), lambda i: (0, 0)),
            pl.BlockSpec((1, DD), lambda i: (0, 0)),
        ],
        out_specs=[
            pl.BlockSpec((BLK, DD), lambda i: (i, 0)),
            pl.BlockSpec((8, DD), lambda i: (0, 0)),
        ],
        out_shape=[
            jax.ShapeDtypeStruct((NP, DD), jnp.float32),
            jax.ShapeDtypeStruct((8, DD), jnp.float32),
        ],
        scratch_shapes=[pltpu.VMEM((8, DD), jnp.float32)],
    )(a0, a1, y, d0, d1, b, gamma, beta)


def _p8_body(h_ref, bat_ref, ss_ref, out_ref):
    i = pl.program_id(0)

    @pl.when(i == 0)
    def _():
        out_ref[...] = jnp.full((NG, DD), -jnp.inf, jnp.float32)

    hn = h_ref[...] * ss_ref[0:1, :] + ss_ref[1:2, :]
    bat = bat_ref[...]

    def body(g, _):
        mask = bat == g
        mm = jnp.max(jnp.where(mask, hn, -jnp.inf), axis=0, keepdims=True)
        cur = out_ref[pl.ds(g, 1), :]
        out_ref[pl.ds(g, 1), :] = jnp.maximum(cur, mm)
        return 0

    lax.fori_loop(0, NG, body, 0)


def _p8(h3, batp, ss):
    return pl.pallas_call(
        _p8_body,
        grid=(NBLK,),
        in_specs=[
            pl.BlockSpec((BLK, DD), lambda i: (i, 0)),
            pl.BlockSpec((BLK, 1), lambda i: (i, 0)),
            pl.BlockSpec((8, DD), lambda i: (0, 0)),
        ],
        out_specs=pl.BlockSpec((NG, DD), lambda i: (0, 0)),
        out_shape=jax.ShapeDtypeStruct((NG, DD), jnp.float32),
    )(h3, batp, ss)


def _p9_body(g_ref, w1_ref, b1_ref, w2_ref, b2_ref, w3_ref, b3_ref, out_ref):
    g = g_ref[...]
    g = jnp.where(g == -jnp.inf, 0.0, g)
    g = jax.nn.relu(_dot(g, w1_ref[...]) + b1_ref[...])
    g = jax.nn.relu(_dot(g, w2_ref[...]) + b2_ref[...])
    out_ref[...] = _dot(g, w3_ref[...]) + b3_ref[...]


def _p9(gmax, lW1, lb1, lW2p, lb2p, lW3p, lb3p):
    return pl.pallas_call(
        _p9_body,
        out_shape=jax.ShapeDtypeStruct((NG, DD), jnp.float32),
    )(gmax, lW1, lb1, lW2p, lb2p, lW3p, lb3p)


def kernel(x, edge_index, batch, W1, b1, W2, b2, W3, b3, gamma, beta,
           lW1, lb1, lW2, lb2, lW3, lb3):
    src = edge_index[0]
    dst = edge_index[1]
    npad = EPAD - EE
    sidx2 = jnp.concatenate([src, jnp.zeros((npad,), jnp.int32)]).reshape(
        NS, CPS, KCH)
    didx2 = jnp.concatenate([dst, jnp.full((npad,), NP - 1, jnp.int32)]).reshape(
        NS, CPS, KCH)
    eidx2 = jnp.concatenate([sidx2, didx2], axis=1)
    sw = jnp.zeros((NS, NC, CAM, KCH), jnp.int32)
    dw = jnp.full((NS, NC, CAM, KCH), NP - 1, jnp.int32)
    sw = sw.at[:, 0, :CA].set(sidx2[:, :CA]).at[:, 1, :CB].set(sidx2[:, CA:])
    dw = dw.at[:, 0, :CA].set(didx2[:, :CA]).at[:, 1, :CB].set(didx2[:, CA:])
    eidxw = jnp.concatenate([sw, dw], axis=2).reshape(NW, 2 * CAM, KCH)
    x_pad = jnp.concatenate([x, jnp.zeros((NP - NN, DD), jnp.float32)], axis=0)
    batp = jnp.concatenate([batch, jnp.full((NP - NN,), NG, jnp.int32)]).reshape(
        NP, 1)

    b1r = b1.reshape(1, DD)
    b2r = b2.reshape(1, DD)
    b3r = b3.reshape(1, DD)
    gr = gamma.reshape(1, DD)
    ber = beta.reshape(1, DD)
    lb1r = lb1.reshape(1, DD)
    lW2p = jnp.pad(lW2, ((0, 0), (0, DD - lW2.shape[1])))
    lb2p = jnp.pad(lb2, (0, DD - lb2.shape[0])).reshape(1, DD)
    lW3p = jnp.pad(lW3, ((0, DD - lW3.shape[0]), (0, DD - lW3.shape[1])))
    lb3p = jnp.pad(lb3, (0, DD - lb3.shape[0])).reshape(1, DD)

    degp = _deg_sc(eidx2)
    d0 = lax.slice(degp[0], (0, 0), (NP, 1))
    d1 = lax.slice(degp[1], (0, 0), (NP, 1))

    y1 = _p1(x_pad, d0, d1, W1)
    a1 = _edge_sc(y1, eidxw)
    y2 = _p3(a1[0], a1[1], y1, d0, d1, b1r, W2)
    a2 = _edge_sc(y2, eidxw)
    y3 = _p3(a2[0], a2[1], y2, d0, d1, b2r, W3)
    a3 = _edge_sc(y3, eidxw)
    h3, ss = _p7(a3[0], a3[1], y3, d0, d1, b3r, gr, ber)
    gmax = _p8(h3, batp, ss)
    out = _p9(gmax, lW1, lb1r, lW2p, lb2p, lW3p, lb3p)
    return out[:, :NCLS]


# speedup vs baseline: 1.3760x; 1.0852x over previous
import functools
import jax
import jax.numpy as jnp
from jax import lax
from jax.experimental import pallas as pl
from jax.experimental.pallas import tpu as pltpu
from jax.experimental.pallas import tpu_sc as plsc

NN = 10000
EE = 320000
DD = 128
NG = 64
NCLS = 10

NC = 2
NS = 16
NW = NC * NS
KCH = 128
CPS = 160
TOTCH = NS * CPS
EPAD = TOTCH * KCH
CA = 120
CB = CPS - CA
CDG = 80
CAM = max(CA, CB)
NP = 10240
RPT = NP // NS

_mesh = plsc.VectorSubcoreMesh(core_axis_name="c", subcore_axis_name="s")


def _zero_rows(ref, nrows, ncolchunks, val=0.0):
    v = jnp.full((16,), val, jnp.float32)

    def body(r, _):
        for k in range(ncolchunks):
            ref[r, pl.ds(k * 16, 16)] = v
        return 0

    lax.fori_loop(0, nrows, body, 0)


@functools.partial(
    pl.kernel,
    out_type=jax.ShapeDtypeStruct((NC, NP, DD), jnp.float32),
    mesh=_mesh,
    scratch_types=[
        pltpu.VMEM((CPS, KCH), jnp.int32),
        pltpu.VMEM((KCH, DD), jnp.float32),
        pltpu.VMEM_SHARED((NP, DD), jnp.float32),
    ],
)
def _deg_sc(eidx_hbm, degp_hbm, didx_v, ones_v, acc_sh):
    cid = lax.axis_index("c")
    sid = lax.axis_index("s")

    _zero_rows(ones_v, KCH, DD // 16, 0.0)

    def zbody(c, _):
        pltpu.sync_copy(ones_v, acc_sh.at[pl.ds(sid * RPT + c * KCH, KCH)])
        return 0

    lax.fori_loop(0, RPT // KCH, zbody, 0)
    plsc.subcore_barrier()

    _zero_rows(ones_v, KCH, DD // 16, 1.0)
    off = cid * CDG
    n = jnp.where(cid == 0, CDG, CPS - CDG)
    pltpu.sync_copy(eidx_hbm.at[sid, pl.ds(CPS, CPS)], didx_v)

    def body(c, _):
        pltpu.sync_copy(ones_v, acc_sh.at[didx_v.at[off + c]], add=True)
        return 0

    lax.fori_loop(0, n, body, 0)
    plsc.subcore_barrier()
    pltpu.sync_copy(acc_sh.at[pl.ds(sid * RPT, RPT)],
                    degp_hbm.at[cid, pl.ds(sid * RPT, RPT)])


@functools.partial(
    pl.kernel,
    out_type=jax.ShapeDtypeStruct((NC, NP, DD), jnp.float32),
    mesh=_mesh,
    scratch_types=[
        pltpu.VMEM((2 * CAM, KCH), jnp.int32),
        pltpu.VMEM((KCH, DD), jnp.float32),
        pltpu.SemaphoreType.DMA,
        pltpu.VMEM_SHARED((NP, DD), jnp.float32),
    ],
)
def _edge_sc(y_hbm, eidx_hbm, accp_hbm, eidx_v, rows_v, sem, acc_sh):
    cid = lax.axis_index("c")
    sid = lax.axis_index("s")

    _zero_rows(rows_v, KCH, DD // 16, 0.0)

    def zbody(c, _):
        pltpu.sync_copy(rows_v, acc_sh.at[pl.ds(sid * RPT + c * KCH, KCH)])
        return 0

    lax.fori_loop(0, RPT // KCH, zbody, 0)
    plsc.subcore_barrier()

    wid = sid * NC + cid
    n = jnp.where(cid == 0, CA, CB)
    pltpu.sync_copy(eidx_hbm.at[wid], eidx_v)

    def body(c, _):
        pltpu.async_copy(y_hbm.at[eidx_v.at[c]], rows_v, sem).wait()
        pltpu.sync_copy(rows_v, acc_sh.at[eidx_v.at[CAM + c]], add=True)
        return 0

    lax.fori_loop(0, n, body, 0)
    plsc.subcore_barrier()
    pltpu.sync_copy(acc_sh.at[pl.ds(sid * RPT, RPT)],
                    accp_hbm.at[cid, pl.ds(sid * RPT, RPT)])


def _dot(a, b):
    return lax.dot_general(a, b, (((1,), (0,)), ((), ())),
                           precision=lax.Precision.HIGHEST,
                           preferred_element_type=jnp.float32)


def _dinv(d0, d1):
    return lax.rsqrt(d0 + d1 + 1.0)


BLK = 1024
NBLK = NP // BLK


def _p1_body(x_ref, d0_ref, d1_ref, w_ref, y_ref):
    dinv = _dinv(d0_ref[...], d1_ref[...])
    y_ref[...] = _dot(x_ref[...], w_ref[...]) * dinv


def _p1(x_pad, d0, d1, W1):
    return pl.pallas_call(
        _p1_body,
        grid=(NBLK,),
        in_specs=[
            pl.BlockSpec((BLK, DD), lambda i: (i, 0)),
            pl.BlockSpec((BLK, 1), lambda i: (i, 0)),
            pl.BlockSpec((BLK, 1), lambda i: (i, 0)),
            pl.BlockSpec((DD, DD), lambda i: (0, 0)),
        ],
        out_specs=pl.BlockSpec((BLK, DD), lambda i: (i, 0)),
        out_shape=jax.ShapeDtypeStruct((NP, DD), jnp.float32),
    )(x_pad, d0, d1, W1)


def _p3_body(a0_ref, a1_ref, y_ref, d0_ref, d1_ref, b_ref, w_ref, out_ref):
    dinv = _dinv(d0_ref[...], d1_ref[...])
    h = jax.nn.relu(dinv * (a0_ref[...] + a1_ref[...] + y_ref[...]) + b_ref[...])
    out_ref[...] = _dot(h, w_ref[...]) * dinv


def _p3(a0, a1, y, d0, d1, b, Wn):
    return pl.pallas_call(
        _p3_body,
        grid=(NBLK,),
        in_specs=[
            pl.BlockSpec((BLK, DD), lambda i: (i, 0)),
            pl.BlockSpec((BLK, DD), lambda i: (i, 0)),
            pl.BlockSpec((BLK, DD), lambda i: (i, 0)),
            pl.BlockSpec((BLK, 1), lambda i: (i, 0)),
            pl.BlockSpec((BLK, 1), lambda i: (i, 0)),
            pl.BlockSpec((1, DD), lambda i: (0, 0)),
            pl.BlockSpec((DD, DD), lambda i: (0, 0)),
        ],
        out_specs=pl.BlockSpec((BLK, DD), lambda i: (i, 0)),
        out_shape=jax.ShapeDtypeStruct((NP, DD), jnp.float32),
    )(a0, a1, y, d0, d1, b, Wn)


def _p7_body(a0_ref, a1_ref, y_ref, d0_ref, d1_ref, b_ref, g_ref, be_ref,
             h_ref, ss_ref, acc_ref):
    i = pl.program_id(0)
    dinv = _dinv(d0_ref[...], d1_ref[...])
    h = jax.nn.relu(dinv * (a0_ref[...] + a1_ref[...] + y_ref[...]) + b_ref[...])
    h_ref[...] = h

    rid = i * BLK + lax.broadcasted_iota(jnp.int32, (BLK, 1), 0)
    hm = jnp.where(rid < NN, h, 0.0)

    @pl.when(i == 0)
    def _():
        acc_ref[...] = jnp.zeros_like(acc_ref)

    acc_ref[0:1, :] += jnp.sum(hm, axis=0, keepdims=True)
    acc_ref[1:2, :] += jnp.sum(hm * hm, axis=0, keepdims=True)

    @pl.when(i == NBLK - 1)
    def _():
        mean = acc_ref[0:1, :] / NN
        var = acc_ref[1:2, :] / NN - mean * mean
        scale = g_ref[...] / jnp.sqrt(var + 1e-5)
        shift = be_ref[...] - mean * scale
        ss_ref[...] = jnp.concatenate(
            [scale, shift, jnp.zeros((6, DD), jnp.float32)], axis=0)


def _p7(a0, a1, y, d0, d1, b, gamma, beta):
    return pl.pallas_call(
        _p7_body,
        grid=(NBLK,),
        in_specs=[
            pl.BlockSpec((BLK, DD), lambda i: (i, 0)),
            pl.BlockSpec((BLK, DD), lambda i: (i, 0)),
            pl.BlockSpec((BLK, DD), lambda i: (i, 0)),
            pl.BlockSpec((BLK, 1), lambda i: (i, 0)),
            pl.BlockSpec((BLK, 1), lambda i: (i, 0)),
            pl.BlockSpec((1, DD), lambda i: (0, 0)),
            pl.BlockSpec((1, DD), lambda i: (0, 0)),
            pl.BlockSpec((1, DD), lambda i: (0, 0)),
        ],
        out_specs=[
            pl.BlockSpec((BLK, DD), lambda i: (i, 0)),
            pl.BlockSpec((8, DD), lambda i: (0, 0)),
        ],
        out_shape=[
            jax.ShapeDtypeStruct((NP, DD), jnp.float32),
            jax.ShapeDtypeStruct((8, DD), jnp.float32),
        ],
        scratch_shapes=[pltpu.VMEM((8, DD), jnp.float32)],
    )(a0, a1, y, d0, d1, b, gamma, beta)


def _p8_body(bat_smem, h_ref, bat_ref, ss_ref, out_ref):
    i = pl.program_id(0)

    @pl.when(i == 0)
    def _():
        out_ref[...] = jnp.full((NG, DD), -jnp.inf, jnp.float32)

    hn = h_ref[...] * ss_ref[0:1, :] + ss_ref[1:2, :]
    bat = bat_ref[...]

    glo = bat_smem[i * BLK]
    ghi = jnp.minimum(bat_smem[i * BLK + BLK - 1], NG - 1)

    def body(g, _):
        mask = bat == g
        mm = jnp.max(jnp.where(mask, hn, -jnp.inf), axis=0, keepdims=True)
        cur = out_ref[pl.ds(g, 1), :]
        out_ref[pl.ds(g, 1), :] = jnp.maximum(cur, mm)
        return 0

    lax.fori_loop(glo, ghi + 1, body, 0)


def _p8(h3, batp, ss):
    return pl.pallas_call(
        _p8_body,
        grid_spec=pltpu.PrefetchScalarGridSpec(
            num_scalar_prefetch=1,
            grid=(NBLK,),
            in_specs=[
                pl.BlockSpec((BLK, DD), lambda i, s: (i, 0)),
                pl.BlockSpec((BLK, 1), lambda i, s: (i, 0)),
                pl.BlockSpec((8, DD), lambda i, s: (0, 0)),
            ],
            out_specs=pl.BlockSpec((NG, DD), lambda i, s: (0, 0)),
        ),
        out_shape=jax.ShapeDtypeStruct((NG, DD), jnp.float32),
    )(batp.reshape(NP), h3, batp, ss)


def _p9_body(g_ref, w1_ref, b1_ref, w2_ref, b2_ref, w3_ref, b3_ref, out_ref):
    g = g_ref[...]
    g = jnp.where(g == -jnp.inf, 0.0, g)
    g = jax.nn.relu(_dot(g, w1_ref[...]) + b1_ref[...])
    g = jax.nn.relu(_dot(g, w2_ref[...]) + b2_ref[...])
    out_ref[...] = _dot(g, w3_ref[...]) + b3_ref[...]


def _p9(gmax, lW1, lb1, lW2p, lb2p, lW3p, lb3p):
    return pl.pallas_call(
        _p9_body,
        out_shape=jax.ShapeDtypeStruct((NG, DD), jnp.float32),
    )(gmax, lW1, lb1, lW2p, lb2p, lW3p, lb3p)


def kernel(x, edge_index, batch, W1, b1, W2, b2, W3, b3, gamma, beta,
           lW1, lb1, lW2, lb2, lW3, lb3):
    src = edge_index[0]
    dst = edge_index[1]
    npad = EPAD - EE
    sidx2 = jnp.concatenate([src, jnp.zeros((npad,), jnp.int32)]).reshape(
        NS, CPS, KCH)
    didx2 = jnp.concatenate([dst, jnp.full((npad,), NP - 1, jnp.int32)]).reshape(
        NS, CPS, KCH)
    eidx2 = jnp.concatenate([sidx2, didx2], axis=1)
    sw = jnp.zeros((NS, NC, CAM, KCH), jnp.int32)
    dw = jnp.full((NS, NC, CAM, KCH), NP - 1, jnp.int32)
    sw = sw.at[:, 0, :CA].set(sidx2[:, :CA]).at[:, 1, :CB].set(sidx2[:, CA:])
    dw = dw.at[:, 0, :CA].set(didx2[:, :CA]).at[:, 1, :CB].set(didx2[:, CA:])
    eidxw = jnp.concatenate([sw, dw], axis=2).reshape(NW, 2 * CAM, KCH)
    x_pad = jnp.concatenate([x, jnp.zeros((NP - NN, DD), jnp.float32)], axis=0)
    batp = jnp.concatenate([batch, jnp.full((NP - NN,), NG, jnp.int32)]).reshape(
        NP, 1)

    b1r = b1.reshape(1, DD)
    b2r = b2.reshape(1, DD)
    b3r = b3.reshape(1, DD)
    gr = gamma.reshape(1, DD)
    ber = beta.reshape(1, DD)
    lb1r = lb1.reshape(1, DD)
    lW2p = jnp.pad(lW2, ((0, 0), (0, DD - lW2.shape[1])))
    lb2p = jnp.pad(lb2, (0, DD - lb2.shape[0])).reshape(1, DD)
    lW3p = jnp.pad(lW3, ((0, DD - lW3.shape[0]), (0, DD - lW3.shape[1])))
    lb3p = jnp.pad(lb3, (0, DD - lb3.shape[0])).reshape(1, DD)

    degp = _deg_sc(eidx2)
    d0 = lax.slice(degp[0], (0, 0), (NP, 1))
    d1 = lax.slice(degp[1], (0, 0), (NP, 1))

    y1 = _p1(x_pad, d0, d1, W1)
    a1 = _edge_sc(y1, eidxw)
    y2 = _p3(a1[0], a1[1], y1, d0, d1, b1r, W2)
    a2 = _edge_sc(y2, eidxw)
    y3 = _p3(a2[0], a2[1], y2, d0, d1, b2r, W3)
    a3 = _edge_sc(y3, eidxw)
    h3, ss = _p7(a3[0], a3[1], y3, d0, d1, b3r, gr, ber)
    gmax = _p8(h3, batp, ss)
    out = _p9(gmax, lW1, lb1r, lW2p, lb2p, lW3p, lb3p)
    return out[:, :NCLS]


# speedup vs baseline: 1.3982x; 1.0162x over previous
import functools
import jax
import jax.numpy as jnp
from jax import lax
from jax.experimental import pallas as pl
from jax.experimental.pallas import tpu as pltpu
from jax.experimental.pallas import tpu_sc as plsc

NN = 10000
EE = 320000
DD = 128
NG = 64
NCLS = 10

NC = 2
NS = 16
NW = NC * NS
KCH = 128
CPS = 160
TOTCH = NS * CPS
EPAD = TOTCH * KCH
CA = 117
CB = CPS - CA
CDG = 80
CAM = max(CA, CB)
NP = 10240
RPT = NP // NS

_mesh = plsc.VectorSubcoreMesh(core_axis_name="c", subcore_axis_name="s")


def _zero_rows(ref, nrows, ncolchunks, val=0.0):
    v = jnp.full((16,), val, jnp.float32)

    def body(r, _):
        for k in range(ncolchunks):
            ref[r, pl.ds(k * 16, 16)] = v
        return 0

    lax.fori_loop(0, nrows, body, 0)


@functools.partial(
    pl.kernel,
    out_type=jax.ShapeDtypeStruct((NC, NP, DD), jnp.float32),
    mesh=_mesh,
    scratch_types=[
        pltpu.VMEM((CPS, KCH), jnp.int32),
        pltpu.VMEM((KCH, DD), jnp.float32),
        pltpu.VMEM_SHARED((NP, DD), jnp.float32),
    ],
)
def _deg_sc(eidx_hbm, degp_hbm, didx_v, ones_v, acc_sh):
    cid = lax.axis_index("c")
    sid = lax.axis_index("s")

    _zero_rows(ones_v, KCH, DD // 16, 0.0)

    def zbody(c, _):
        pltpu.sync_copy(ones_v, acc_sh.at[pl.ds(sid * RPT + c * KCH, KCH)])
        return 0

    lax.fori_loop(0, RPT // KCH, zbody, 0)
    plsc.subcore_barrier()

    _zero_rows(ones_v, KCH, DD // 16, 1.0)
    off = cid * CDG
    n = jnp.where(cid == 0, CDG, CPS - CDG)
    pltpu.sync_copy(eidx_hbm.at[sid, pl.ds(CPS, CPS)], didx_v)

    def body(c, _):
        pltpu.sync_copy(ones_v, acc_sh.at[didx_v.at[off + c]], add=True)
        return 0

    lax.fori_loop(0, n, body, 0)
    plsc.subcore_barrier()
    pltpu.sync_copy(acc_sh.at[pl.ds(sid * RPT, RPT)],
                    degp_hbm.at[cid, pl.ds(sid * RPT, RPT)])


@functools.partial(
    pl.kernel,
    out_type=jax.ShapeDtypeStruct((NC, NP, DD), jnp.float32),
    mesh=_mesh,
    scratch_types=[
        pltpu.VMEM((2 * CAM, KCH), jnp.int32),
        pltpu.VMEM((KCH, DD), jnp.float32),
        pltpu.SemaphoreType.DMA,
        pltpu.VMEM_SHARED((NP, DD), jnp.float32),
    ],
)
def _edge_sc(y_hbm, eidx_hbm, accp_hbm, eidx_v, rows_v, sem, acc_sh):
    cid = lax.axis_index("c")
    sid = lax.axis_index("s")

    _zero_rows(rows_v, KCH, DD // 16, 0.0)

    def zbody(c, _):
        pltpu.sync_copy(rows_v, acc_sh.at[pl.ds(sid * RPT + c * KCH, KCH)])
        return 0

    lax.fori_loop(0, RPT // KCH, zbody, 0)
    plsc.subcore_barrier()

    wid = sid * NC + cid
    n = jnp.where(cid == 0, CA, CB)
    pltpu.sync_copy(eidx_hbm.at[wid], eidx_v)

    def body(c, _):
        pltpu.async_copy(y_hbm.at[eidx_v.at[c]], rows_v, sem).wait()
        pltpu.sync_copy(rows_v, acc_sh.at[eidx_v.at[CAM + c]], add=True)
        return 0

    lax.fori_loop(0, n, body, 0)
    plsc.subcore_barrier()
    pltpu.sync_copy(acc_sh.at[pl.ds(sid * RPT, RPT)],
                    accp_hbm.at[cid, pl.ds(sid * RPT, RPT)])


def _dot(a, b):
    return lax.dot_general(a, b, (((1,), (0,)), ((), ())),
                           precision=lax.Precision.HIGHEST,
                           preferred_element_type=jnp.float32)


def _dinv(d0, d1):
    return lax.rsqrt(d0 + d1 + 1.0)


BLK = 1024
NBLK = NP // BLK


def _p1_body(x_ref, d0_ref, d1_ref, w_ref, y_ref):
    dinv = _dinv(d0_ref[...], d1_ref[...])
    y_ref[...] = _dot(x_ref[...], w_ref[...]) * dinv


def _p1(x_pad, d0, d1, W1):
    return pl.pallas_call(
        _p1_body,
        grid=(NBLK,),
        in_specs=[
            pl.BlockSpec((BLK, DD), lambda i: (i, 0)),
            pl.BlockSpec((BLK, 1), lambda i: (i, 0)),
            pl.BlockSpec((BLK, 1), lambda i: (i, 0)),
            pl.BlockSpec((DD, DD), lambda i: (0, 0)),
        ],
        out_specs=pl.BlockSpec((BLK, DD), lambda i: (i, 0)),
        out_shape=jax.ShapeDtypeStruct((NP, DD), jnp.float32),
    )(x_pad, d0, d1, W1)


def _p3_body(a0_ref, a1_ref, y_ref, d0_ref, d1_ref, b_ref, w_ref, out_ref):
    dinv = _dinv(d0_ref[...], d1_ref[...])
    h = jax.nn.relu(dinv * (a0_ref[...] + a1_ref[...] + y_ref[...]) + b_ref[...])
    out_ref[...] = _dot(h, w_ref[...]) * dinv


def _p3(a0, a1, y, d0, d1, b, Wn):
    return pl.pallas_call(
        _p3_body,
        grid=(NBLK,),
        in_specs=[
            pl.BlockSpec((BLK, DD), lambda i: (i, 0)),
            pl.BlockSpec((BLK, DD), lambda i: (i, 0)),
            pl.BlockSpec((BLK, DD), lambda i: (i, 0)),
            pl.BlockSpec((BLK, 1), lambda i: (i, 0)),
            pl.BlockSpec((BLK, 1), lambda i: (i, 0)),
            pl.BlockSpec((1, DD), lambda i: (0, 0)),
            pl.BlockSpec((DD, DD), lambda i: (0, 0)),
        ],
        out_specs=pl.BlockSpec((BLK, DD), lambda i: (i, 0)),
        out_shape=jax.ShapeDtypeStruct((NP, DD), jnp.float32),
    )(a0, a1, y, d0, d1, b, Wn)


def _p7_body(a0_ref, a1_ref, y_ref, d0_ref, d1_ref, b_ref, g_ref, be_ref,
             h_ref, ss_ref, acc_ref):
    i = pl.program_id(0)
    dinv = _dinv(d0_ref[...], d1_ref[...])
    h = jax.nn.relu(dinv * (a0_ref[...] + a1_ref[...] + y_ref[...]) + b_ref[...])
    h_ref[...] = h

    rid = i * BLK + lax.broadcasted_iota(jnp.int32, (BLK, 1), 0)
    hm = jnp.where(rid < NN, h, 0.0)

    @pl.when(i == 0)
    def _():
        acc_ref[...] = jnp.zeros_like(acc_ref)

    acc_ref[0:1, :] += jnp.sum(hm, axis=0, keepdims=True)
    acc_ref[1:2, :] += jnp.sum(hm * hm, axis=0, keepdims=True)

    @pl.when(i == NBLK - 1)
    def _():
        mean = acc_ref[0:1, :] / NN
        var = acc_ref[1:2, :] / NN - mean * mean
        scale = g_ref[...] / jnp.sqrt(var + 1e-5)
        shift = be_ref[...] - mean * scale
        ss_ref[...] = jnp.concatenate(
            [scale, shift, jnp.zeros((6, DD), jnp.float32)], axis=0)


def _p7(a0, a1, y, d0, d1, b, gamma, beta):
    return pl.pallas_call(
        _p7_body,
        grid=(NBLK,),
        in_specs=[
            pl.BlockSpec((BLK, DD), lambda i: (i, 0)),
            pl.BlockSpec((BLK, DD), lambda i: (i, 0)),
            pl.BlockSpec((BLK, DD), lambda i: (i, 0)),
            pl.BlockSpec((BLK, 1), lambda i: (i, 0)),
            pl.BlockSpec((BLK, 1), lambda i: (i, 0)),
            pl.BlockSpec((1, DD), lambda i: (0, 0)),
            pl.BlockSpec((1, DD), lambda i: (0, 0)),
            pl.BlockSpec((1, DD), lambda i: (0, 0)),
        ],
        out_specs=[
            pl.BlockSpec((BLK, DD), lambda i: (i, 0)),
            pl.BlockSpec((8, DD), lambda i: (0, 0)),
        ],
        out_shape=[
            jax.ShapeDtypeStruct((NP, DD), jnp.float32),
            jax.ShapeDtypeStruct((8, DD), jnp.float32),
        ],
        scratch_shapes=[pltpu.VMEM((8, DD), jnp.float32)],
    )(a0, a1, y, d0, d1, b, gamma, beta)


def _p8_body(bat_smem, h_ref, bat_ref, ss_ref, out_ref):
    i = pl.program_id(0)

    @pl.when(i == 0)
    def _():
        out_ref[...] = jnp.full((NG, DD), -jnp.inf, jnp.float32)

    hn = h_ref[...] * ss_ref[0:1, :] + ss_ref[1:2, :]
    bat = bat_ref[...]

    glo = bat_smem[i * BLK]
    ghi = jnp.minimum(bat_smem[i * BLK + BLK - 1], NG - 1)

    def body(g, _):
        mask = bat == g
        mm = jnp.max(jnp.where(mask, hn, -jnp.inf), axis=0, keepdims=True)
        cur = out_ref[pl.ds(g, 1), :]
        out_ref[pl.ds(g, 1), :] = jnp.maximum(cur, mm)
        return 0

    lax.fori_loop(glo, ghi + 1, body, 0)


def _p8(h3, batp, ss):
    return pl.pallas_call(
        _p8_body,
        grid_spec=pltpu.PrefetchScalarGridSpec(
            num_scalar_prefetch=1,
            grid=(NBLK,),
            in_specs=[
                pl.BlockSpec((BLK, DD), lambda i, s: (i, 0)),
                pl.BlockSpec((BLK, 1), lambda i, s: (i, 0)),
                pl.BlockSpec((8, DD), lambda i, s: (0, 0)),
            ],
            out_specs=pl.BlockSpec((NG, DD), lambda i, s: (0, 0)),
        ),
        out_shape=jax.ShapeDtypeStruct((NG, DD), jnp.float32),
    )(batp.reshape(NP), h3, batp, ss)


def _p9_body(g_ref, w1_ref, b1_ref, w2_ref, b2_ref, w3_ref, b3_ref, out_ref):
    g = g_ref[...]
    g = jnp.where(g == -jnp.inf, 0.0, g)
    g = jax.nn.relu(_dot(g, w1_ref[...]) + b1_ref[...])
    g = jax.nn.relu(_dot(g, w2_ref[...]) + b2_ref[...])
    out_ref[...] = _dot(g, w3_ref[...]) + b3_ref[...]


def _p9(gmax, lW1, lb1, lW2p, lb2p, lW3p, lb3p):
    return pl.pallas_call(
        _p9_body,
        out_shape=jax.ShapeDtypeStruct((NG, DD), jnp.float32),
    )(gmax, lW1, lb1, lW2p, lb2p, lW3p, lb3p)


def kernel(x, edge_index, batch, W1, b1, W2, b2, W3, b3, gamma, beta,
           lW1, lb1, lW2, lb2, lW3, lb3):
    src = edge_index[0]
    dst = edge_index[1]
    npad = EPAD - EE
    sidx2 = jnp.concatenate([src, jnp.zeros((npad,), jnp.int32)]).reshape(
        NS, CPS, KCH)
    didx2 = jnp.concatenate([dst, jnp.full((npad,), NP - 1, jnp.int32)]).reshape(
        NS, CPS, KCH)
    eidx2 = jnp.concatenate([sidx2, didx2], axis=1)
    sw = jnp.zeros((NS, NC, CAM, KCH), jnp.int32)
    dw = jnp.full((NS, NC, CAM, KCH), NP - 1, jnp.int32)
    sw = sw.at[:, 0, :CA].set(sidx2[:, :CA]).at[:, 1, :CB].set(sidx2[:, CA:])
    dw = dw.at[:, 0, :CA].set(didx2[:, :CA]).at[:, 1, :CB].set(didx2[:, CA:])
    eidxw = jnp.concatenate([sw, dw], axis=2).reshape(NW, 2 * CAM, KCH)
    x_pad = jnp.concatenate([x, jnp.zeros((NP - NN, DD), jnp.float32)], axis=0)
    batp = jnp.concatenate([batch, jnp.full((NP - NN,), NG, jnp.int32)]).reshape(
        NP, 1)

    b1r = b1.reshape(1, DD)
    b2r = b2.reshape(1, DD)
    b3r = b3.reshape(1, DD)
    gr = gamma.reshape(1, DD)
    ber = beta.reshape(1, DD)
    lb1r = lb1.reshape(1, DD)
    lW2p = jnp.pad(lW2, ((0, 0), (0, DD - lW2.shape[1])))
    lb2p = jnp.pad(lb2, (0, DD - lb2.shape[0])).reshape(1, DD)
    lW3p = jnp.pad(lW3, ((0, DD - lW3.shape[0]), (0, DD - lW3.shape[1])))
    lb3p = jnp.pad(lb3, (0, DD - lb3.shape[0])).reshape(1, DD)

    degp = _deg_sc(eidx2)
    d0 = lax.slice(degp[0], (0, 0), (NP, 1))
    d1 = lax.slice(degp[1], (0, 0), (NP, 1))

    y1 = _p1(x_pad, d0, d1, W1)
    a1 = _edge_sc(y1, eidxw)
    y2 = _p3(a1[0], a1[1], y1, d0, d1, b1r, W2)
    a2 = _edge_sc(y2, eidxw)
    y3 = _p3(a2[0], a2[1], y2, d0, d1, b2r, W3)
    a3 = _edge_sc(y3, eidxw)
    h3, ss = _p7(a3[0], a3[1], y3, d0, d1, b3r, gr, ber)
    gmax = _p8(h3, batp, ss)
    out = _p9(gmax, lW1, lb1r, lW2p, lb2p, lW3p, lb3p)
    return out[:, :NCLS]


# speedup vs baseline: 1.4253x; 1.0194x over previous
import functools
import jax
import jax.numpy as jnp
from jax import lax
from jax.experimental import pallas as pl
from jax.experimental.pallas import tpu as pltpu
from jax.experimental.pallas import tpu_sc as plsc

NN = 10000
EE = 320000
DD = 128
NG = 64
NCLS = 10

NC = 2
NS = 16
NW = NC * NS
KCH = 128
CPS = 160
TOTCH = NS * CPS
EPAD = TOTCH * KCH
CA = 117
CB = CPS - CA
CDG = 80
CAM = max(CA, CB)
NP = 10240
RPT = NP // NS

_mesh = plsc.VectorSubcoreMesh(core_axis_name="c", subcore_axis_name="s")


def _zero_rows(ref, nrows, ncolchunks, val=0.0):
    v = jnp.full((16,), val, jnp.float32)

    def body(r, _):
        for k in range(ncolchunks):
            ref[r, pl.ds(k * 16, 16)] = v
        return 0

    lax.fori_loop(0, nrows, body, 0)


@functools.partial(
    pl.kernel,
    out_type=jax.ShapeDtypeStruct((NC, NP, DD), jnp.float32),
    mesh=_mesh,
    scratch_types=[
        pltpu.VMEM((CPS, KCH), jnp.int32),
        pltpu.VMEM((KCH, DD), jnp.float32),
        pltpu.VMEM_SHARED((NP, DD), jnp.float32),
    ],
)
def _deg_sc(eidx_hbm, degp_hbm, didx_v, ones_v, acc_sh):
    cid = lax.axis_index("c")
    sid = lax.axis_index("s")

    _zero_rows(ones_v, KCH, DD // 16, 0.0)

    def zbody(c, _):
        pltpu.sync_copy(ones_v, acc_sh.at[pl.ds(sid * RPT + c * KCH, KCH)])
        return 0

    lax.fori_loop(0, RPT // KCH, zbody, 0)
    plsc.subcore_barrier()

    _zero_rows(ones_v, KCH, DD // 16, 1.0)
    off = cid * CDG
    n = jnp.where(cid == 0, CDG, CPS - CDG)
    pltpu.sync_copy(eidx_hbm.at[sid, pl.ds(CPS, CPS)], didx_v)

    def body(c, _):
        pltpu.sync_copy(ones_v, acc_sh.at[didx_v.at[off + c]], add=True)
        return 0

    lax.fori_loop(0, n, body, 0)
    plsc.subcore_barrier()
    pltpu.sync_copy(acc_sh.at[pl.ds(sid * RPT, RPT)],
                    degp_hbm.at[cid, pl.ds(sid * RPT, RPT)])


@functools.partial(
    pl.kernel,
    out_type=jax.ShapeDtypeStruct((NC, NP, DD), jnp.float32),
    mesh=_mesh,
    scratch_types=[
        pltpu.VMEM((2 * CAM, KCH), jnp.int32),
        pltpu.VMEM((KCH, DD), jnp.float32),
        pltpu.SemaphoreType.DMA,
        pltpu.VMEM_SHARED((NP, DD), jnp.float32),
    ],
)
def _edge_sc(y_hbm, eidx_hbm, accp_hbm, eidx_v, rows_v, sem, acc_sh):
    cid = lax.axis_index("c")
    sid = lax.axis_index("s")

    _zero_rows(rows_v, KCH, DD // 16, 0.0)

    def zbody(c, _):
        pltpu.sync_copy(rows_v, acc_sh.at[pl.ds(sid * RPT + c * KCH, KCH)])
        return 0

    lax.fori_loop(0, RPT // KCH, zbody, 0)
    plsc.subcore_barrier()

    wid = sid * NC + cid
    n = jnp.where(cid == 0, CA, CB)
    pltpu.sync_copy(eidx_hbm.at[wid], eidx_v)

    def body(c, _):
        pltpu.async_copy(y_hbm.at[eidx_v.at[c]], rows_v, sem).wait()
        pltpu.sync_copy(rows_v, acc_sh.at[eidx_v.at[CAM + c]], add=True)
        return 0

    lax.fori_loop(0, n, body, 0)
    plsc.subcore_barrier()
    pltpu.sync_copy(acc_sh.at[pl.ds(sid * RPT, RPT)],
                    accp_hbm.at[cid, pl.ds(sid * RPT, RPT)])


def _dot(a, b):
    return lax.dot_general(a, b, (((1,), (0,)), ((), ())),
                           precision=lax.Precision.HIGHEST,
                           preferred_element_type=jnp.float32)


def _dinv(d0, d1):
    return lax.rsqrt(d0 + d1 + 1.0)


BLK = 1024
NBLK = NP // BLK


def _p1a_body(x_ref, w_ref, y_ref):
    y_ref[...] = _dot(x_ref[...], w_ref[...])


def _p1a(x_pad, W1):
    return pl.pallas_call(
        _p1a_body,
        grid=(NBLK,),
        in_specs=[
            pl.BlockSpec((BLK, DD), lambda i: (i, 0)),
            pl.BlockSpec((DD, DD), lambda i: (0, 0)),
        ],
        out_specs=pl.BlockSpec((BLK, DD), lambda i: (i, 0)),
        out_shape=jax.ShapeDtypeStruct((NP, DD), jnp.float32),
    )(x_pad, W1)


def _p1b_body(xw_ref, d0_ref, d1_ref, y_ref):
    y_ref[...] = xw_ref[...] * _dinv(d0_ref[...], d1_ref[...])


def _p1b(xw, d0, d1):
    return pl.pallas_call(
        _p1b_body,
        grid=(NBLK,),
        in_specs=[
            pl.BlockSpec((BLK, DD), lambda i: (i, 0)),
            pl.BlockSpec((BLK, 1), lambda i: (i, 0)),
            pl.BlockSpec((BLK, 1), lambda i: (i, 0)),
        ],
        out_specs=pl.BlockSpec((BLK, DD), lambda i: (i, 0)),
        out_shape=jax.ShapeDtypeStruct((NP, DD), jnp.float32),
    )(xw, d0, d1)


def _p3_body(a0_ref, a1_ref, y_ref, d0_ref, d1_ref, b_ref, w_ref, out_ref):
    dinv = _dinv(d0_ref[...], d1_ref[...])
    h = jax.nn.relu(dinv * (a0_ref[...] + a1_ref[...] + y_ref[...]) + b_ref[...])
    out_ref[...] = _dot(h, w_ref[...]) * dinv


def _p3(a0, a1, y, d0, d1, b, Wn):
    return pl.pallas_call(
        _p3_body,
        grid=(NBLK,),
        in_specs=[
            pl.BlockSpec((BLK, DD), lambda i: (i, 0)),
            pl.BlockSpec((BLK, DD), lambda i: (i, 0)),
            pl.BlockSpec((BLK, DD), lambda i: (i, 0)),
            pl.BlockSpec((BLK, 1), lambda i: (i, 0)),
            pl.BlockSpec((BLK, 1), lambda i: (i, 0)),
            pl.BlockSpec((1, DD), lambda i: (0, 0)),
            pl.BlockSpec((DD, DD), lambda i: (0, 0)),
        ],
        out_specs=pl.BlockSpec((BLK, DD), lambda i: (i, 0)),
        out_shape=jax.ShapeDtypeStruct((NP, DD), jnp.float32),
    )(a0, a1, y, d0, d1, b, Wn)


def _p7_body(a0_ref, a1_ref, y_ref, d0_ref, d1_ref, b_ref, g_ref, be_ref,
             h_ref, ss_ref, acc_ref):
    i = pl.program_id(0)
    dinv = _dinv(d0_ref[...], d1_ref[...])
    h = jax.nn.relu(dinv * (a0_ref[...] + a1_ref[...] + y_ref[...]) + b_ref[...])
    h_ref[...] = h

    rid = i * BLK + lax.broadcasted_iota(jnp.int32, (BLK, 1), 0)
    hm = jnp.where(rid < NN, h, 0.0)

    @pl.when(i == 0)
    def _():
        acc_ref[...] = jnp.zeros_like(acc_ref)

    acc_ref[0:1, :] += jnp.sum(hm, axis=0, keepdims=True)
    acc_ref[1:2, :] += jnp.sum(hm * hm, axis=0, keepdims=True)

    @pl.when(i == NBLK - 1)
    def _():
        mean = acc_ref[0:1, :] / NN
        var = acc_ref[1:2, :] / NN - mean * mean
        scale = g_ref[...] / jnp.sqrt(var + 1e-5)
        shift = be_ref[...] - mean * scale
        ss_ref[...] = jnp.concatenate(
            [scale, shift, jnp.zeros((6, DD), jnp.float32)], axis=0)


def _p7(a0, a1, y, d0, d1, b, gamma, beta):
    return pl.pallas_call(
        _p7_body,
        grid=(NBLK,),
        in_specs=[
            pl.BlockSpec((BLK, DD), lambda i: (i, 0)),
            pl.BlockSpec((BLK, DD), lambda i: (i, 0)),
            pl.BlockSpec((BLK, DD), lambda i: (i, 0)),
            pl.BlockSpec((BLK, 1), lambda i: (i, 0)),
            pl.BlockSpec((BLK, 1), lambda i: (i, 0)),
            pl.BlockSpec((1, DD), lambda i: (0, 0)),
            pl.BlockSpec((1, DD), lambda i: (0, 0)),
            pl.BlockSpec((1, DD), lambda i: (0, 0)),
        ],
        out_specs=[
            pl.BlockSpec((BLK, DD), lambda i: (i, 0)),
            pl.BlockSpec((8, DD), lambda i: (0, 0)),
        ],
        out_shape=[
            jax.ShapeDtypeStruct((NP, DD), jnp.float32),
            jax.ShapeDtypeStruct((8, DD), jnp.float32),
        ],
        scratch_shapes=[pltpu.VMEM((8, DD), jnp.float32)],
    )(a0, a1, y, d0, d1, b, gamma, beta)


def _p8_body(bat_smem, h_ref, bat_ref, ss_ref, w1_ref, b1_ref, w2_ref,
             b2_ref, w3_ref, b3_ref, out_ref, gmax_ref):
    i = pl.program_id(0)

    @pl.when(i == 0)
    def _():
        gmax_ref[...] = jnp.full((NG, DD), -jnp.inf, jnp.float32)

    hn = h_ref[...] * ss_ref[0:1, :] + ss_ref[1:2, :]
    bat = bat_ref[...]

    glo = bat_smem[i * BLK]
    ghi = jnp.minimum(bat_smem[i * BLK + BLK - 1], NG - 1)

    def body(g, _):
        mask = bat == g
        mm = jnp.max(jnp.where(mask, hn, -jnp.inf), axis=0, keepdims=True)
        cur = gmax_ref[pl.ds(g, 1), :]
        gmax_ref[pl.ds(g, 1), :] = jnp.maximum(cur, mm)
        return 0

    lax.fori_loop(glo, ghi + 1, body, 0)

    @pl.when(i == NBLK - 1)
    def _():
        g = gmax_ref[...]
        g = jnp.where(g == -jnp.inf, 0.0, g)
        g = jax.nn.relu(_dot(g, w1_ref[...]) + b1_ref[...])
        g = jax.nn.relu(_dot(g, w2_ref[...]) + b2_ref[...])
        out_ref[...] = _dot(g, w3_ref[...]) + b3_ref[...]


def _p8(h3, batp, ss, lW1, lb1, lW2p, lb2p, lW3p, lb3p):
    full = lambda i, s: (0, 0)
    out, _ = pl.pallas_call(
        _p8_body,
        grid_spec=pltpu.PrefetchScalarGridSpec(
            num_scalar_prefetch=1,
            grid=(NBLK,),
            in_specs=[
                pl.BlockSpec((BLK, DD), lambda i, s: (i, 0)),
                pl.BlockSpec((BLK, 1), lambda i, s: (i, 0)),
                pl.BlockSpec((8, DD), full),
                pl.BlockSpec((DD, DD), full),
                pl.BlockSpec((1, DD), full),
                pl.BlockSpec((DD, DD), full),
                pl.BlockSpec((1, DD), full),
                pl.BlockSpec((DD, DD), full),
                pl.BlockSpec((1, DD), full),
            ],
            out_specs=[
                pl.BlockSpec((NG, DD), full),
                pl.BlockSpec((NG, DD), full),
            ],
        ),
        out_shape=[
            jax.ShapeDtypeStruct((NG, DD), jnp.float32),
            jax.ShapeDtypeStruct((NG, DD), jnp.float32),
        ],
    )(batp.reshape(NP), h3, batp, ss, lW1, lb1, lW2p, lb2p, lW3p, lb3p)
    return out


def kernel(x, edge_index, batch, W1, b1, W2, b2, W3, b3, gamma, beta,
           lW1, lb1, lW2, lb2, lW3, lb3):
    src = edge_index[0]
    dst = edge_index[1]
    npad = EPAD - EE
    sidx2 = jnp.concatenate([src, jnp.zeros((npad,), jnp.int32)]).reshape(
        NS, CPS, KCH)
    didx2 = jnp.concatenate([dst, jnp.full((npad,), NP - 1, jnp.int32)]).reshape(
        NS, CPS, KCH)
    eidx2 = jnp.concatenate([sidx2, didx2], axis=1)
    sw = jnp.zeros((NS, NC, CAM, KCH), jnp.int32)
    dw = jnp.full((NS, NC, CAM, KCH), NP - 1, jnp.int32)
    sw = sw.at[:, 0, :CA].set(sidx2[:, :CA]).at[:, 1, :CB].set(sidx2[:, CA:])
    dw = dw.at[:, 0, :CA].set(didx2[:, :CA]).at[:, 1, :CB].set(didx2[:, CA:])
    eidxw = jnp.concatenate([sw, dw], axis=2).reshape(NW, 2 * CAM, KCH)
    x_pad = jnp.concatenate([x, jnp.zeros((NP - NN, DD), jnp.float32)], axis=0)
    batp = jnp.concatenate([batch, jnp.full((NP - NN,), NG, jnp.int32)]).reshape(
        NP, 1)

    b1r = b1.reshape(1, DD)
    b2r = b2.reshape(1, DD)
    b3r = b3.reshape(1, DD)
    gr = gamma.reshape(1, DD)
    ber = beta.reshape(1, DD)
    lb1r = lb1.reshape(1, DD)
    lW2p = jnp.pad(lW2, ((0, 0), (0, DD - lW2.shape[1])))
    lb2p = jnp.pad(lb2, (0, DD - lb2.shape[0])).reshape(1, DD)
    lW3p = jnp.pad(lW3, ((0, DD - lW3.shape[0]), (0, DD - lW3.shape[1])))
    lb3p = jnp.pad(lb3, (0, DD - lb3.shape[0])).reshape(1, DD)

    degp = _deg_sc(eidx2)
    d0 = lax.slice(degp[0], (0, 0), (NP, 1))
    d1 = lax.slice(degp[1], (0, 0), (NP, 1))

    xw1 = _p1a(x_pad, W1)
    y1 = _p1b(xw1, d0, d1)
    a1 = _edge_sc(y1, eidxw)
    y2 = _p3(a1[0], a1[1], y1, d0, d1, b1r, W2)
    a2 = _edge_sc(y2, eidxw)
    y3 = _p3(a2[0], a2[1], y2, d0, d1, b2r, W3)
    a3 = _edge_sc(y3, eidxw)
    h3, ss = _p7(a3[0], a3[1], y3, d0, d1, b3r, gr, ber)
    out = _p8(h3, batp, ss, lW1, lb1r, lW2p, lb2p, lW3p, lb3p)
    return out[:, :NCLS]


# speedup vs baseline: 1.4410x; 1.0110x over previous
import functools
import jax
import jax.numpy as jnp
from jax import lax
from jax.experimental import pallas as pl
from jax.experimental.pallas import tpu as pltpu
from jax.experimental.pallas import tpu_sc as plsc

NN = 10000
EE = 320000
DD = 128
NG = 64
NCLS = 10

NC = 2
NS = 16
NW = NC * NS
KCH = 128
CPS = 160
TOTCH = NS * CPS
EPAD = TOTCH * KCH
CA = 116
CB = CPS - CA
CDG = 80
CAM = max(CA, CB)
NP = 10240
RPT = NP // NS

_mesh = plsc.VectorSubcoreMesh(core_axis_name="c", subcore_axis_name="s")


def _zero_rows(ref, nrows, ncolchunks, val=0.0):
    v = jnp.full((16,), val, jnp.float32)

    def body(r, _):
        for k in range(ncolchunks):
            ref[r, pl.ds(k * 16, 16)] = v
        return 0

    lax.fori_loop(0, nrows, body, 0)


@functools.partial(
    pl.kernel,
    out_type=jax.ShapeDtypeStruct((NC, NP, DD), jnp.float32),
    mesh=_mesh,
    scratch_types=[
        pltpu.VMEM((CPS, KCH), jnp.int32),
        pltpu.VMEM((KCH, DD), jnp.float32),
        pltpu.VMEM_SHARED((NP, DD), jnp.float32),
    ],
)
def _deg_sc(eidx_hbm, degp_hbm, didx_v, ones_v, acc_sh):
    cid = lax.axis_index("c")
    sid = lax.axis_index("s")

    _zero_rows(ones_v, KCH, DD // 16, 0.0)

    def zbody(c, _):
        pltpu.sync_copy(ones_v, acc_sh.at[pl.ds(sid * RPT + c * KCH, KCH)])
        return 0

    lax.fori_loop(0, RPT // KCH, zbody, 0)
    plsc.subcore_barrier()

    _zero_rows(ones_v, KCH, DD // 16, 1.0)
    off = cid * CDG
    n = jnp.where(cid == 0, CDG, CPS - CDG)
    pltpu.sync_copy(eidx_hbm.at[sid, pl.ds(CPS, CPS)], didx_v)

    def body(c, _):
        pltpu.sync_copy(ones_v, acc_sh.at[didx_v.at[off + c]], add=True)
        return 0

    lax.fori_loop(0, n, body, 0)
    plsc.subcore_barrier()
    pltpu.sync_copy(acc_sh.at[pl.ds(sid * RPT, RPT)],
                    degp_hbm.at[cid, pl.ds(sid * RPT, RPT)])


@functools.partial(
    pl.kernel,
    out_type=jax.ShapeDtypeStruct((NC, NP, DD), jnp.float32),
    mesh=_mesh,
    scratch_types=[
        pltpu.VMEM((2 * CAM, KCH), jnp.int32),
        pltpu.VMEM((KCH, DD), jnp.float32),
        pltpu.SemaphoreType.DMA,
        pltpu.VMEM_SHARED((NP, DD), jnp.float32),
    ],
)
def _edge_sc(y_hbm, eidx_hbm, accp_hbm, eidx_v, rows_v, sem, acc_sh):
    cid = lax.axis_index("c")
    sid = lax.axis_index("s")

    _zero_rows(rows_v, KCH, DD // 16, 0.0)

    def zbody(c, _):
        pltpu.sync_copy(rows_v, acc_sh.at[pl.ds(sid * RPT + c * KCH, KCH)])
        return 0

    lax.fori_loop(0, RPT // KCH, zbody, 0)
    plsc.subcore_barrier()

    wid = sid * NC + cid
    n = jnp.where(cid == 0, CA, CB)
    pltpu.sync_copy(eidx_hbm.at[wid], eidx_v)

    def body(c, _):
        pltpu.async_copy(y_hbm.at[eidx_v.at[c]], rows_v, sem).wait()
        pltpu.sync_copy(rows_v, acc_sh.at[eidx_v.at[CAM + c]], add=True)
        return 0

    lax.fori_loop(0, n, body, 0)
    plsc.subcore_barrier()
    pltpu.sync_copy(acc_sh.at[pl.ds(sid * RPT, RPT)],
                    accp_hbm.at[cid, pl.ds(sid * RPT, RPT)])


def _dot(a, b):
    return lax.dot_general(a, b, (((1,), (0,)), ((), ())),
                           precision=lax.Precision.HIGHEST,
                           preferred_element_type=jnp.float32)


def _dinv(d0, d1):
    return lax.rsqrt(d0 + d1 + 1.0)


BLK = 1024
NBLK = NP // BLK


def _p1a_body(x_ref, w_ref, y_ref):
    y_ref[...] = _dot(x_ref[...], w_ref[...])


def _p1a(x_pad, W1):
    return pl.pallas_call(
        _p1a_body,
        grid=(NBLK,),
        in_specs=[
            pl.BlockSpec((BLK, DD), lambda i: (i, 0)),
            pl.BlockSpec((DD, DD), lambda i: (0, 0)),
        ],
        out_specs=pl.BlockSpec((BLK, DD), lambda i: (i, 0)),
        out_shape=jax.ShapeDtypeStruct((NP, DD), jnp.float32),
    )(x_pad, W1)


def _p1b_body(xw_ref, d0_ref, d1_ref, y_ref):
    y_ref[...] = xw_ref[...] * _dinv(d0_ref[...], d1_ref[...])


def _p1b(xw, d0, d1):
    return pl.pallas_call(
        _p1b_body,
        grid=(NBLK,),
        in_specs=[
            pl.BlockSpec((BLK, DD), lambda i: (i, 0)),
            pl.BlockSpec((BLK, 1), lambda i: (i, 0)),
            pl.BlockSpec((BLK, 1), lambda i: (i, 0)),
        ],
        out_specs=pl.BlockSpec((BLK, DD), lambda i: (i, 0)),
        out_shape=jax.ShapeDtypeStruct((NP, DD), jnp.float32),
    )(xw, d0, d1)


def _p3_body(a0_ref, a1_ref, y_ref, d0_ref, d1_ref, b_ref, w_ref, out_ref):
    dinv = _dinv(d0_ref[...], d1_ref[...])
    h = jax.nn.relu(dinv * (a0_ref[...] + a1_ref[...] + y_ref[...]) + b_ref[...])
    out_ref[...] = _dot(h, w_ref[...]) * dinv


def _p3(a0, a1, y, d0, d1, b, Wn):
    return pl.pallas_call(
        _p3_body,
        grid=(NBLK,),
        in_specs=[
            pl.BlockSpec((BLK, DD), lambda i: (i, 0)),
            pl.BlockSpec((BLK, DD), lambda i: (i, 0)),
            pl.BlockSpec((BLK, DD), lambda i: (i, 0)),
            pl.BlockSpec((BLK, 1), lambda i: (i, 0)),
            pl.BlockSpec((BLK, 1), lambda i: (i, 0)),
            pl.BlockSpec((1, DD), lambda i: (0, 0)),
            pl.BlockSpec((DD, DD), lambda i: (0, 0)),
        ],
        out_specs=pl.BlockSpec((BLK, DD), lambda i: (i, 0)),
        out_shape=jax.ShapeDtypeStruct((NP, DD), jnp.float32),
    )(a0, a1, y, d0, d1, b, Wn)


def _p7_body(a0_ref, a1_ref, y_ref, d0_ref, d1_ref, b_ref, g_ref, be_ref,
             h_ref, ss_ref, acc_ref):
    i = pl.program_id(0)
    dinv = _dinv(d0_ref[...], d1_ref[...])
    h = jax.nn.relu(dinv * (a0_ref[...] + a1_ref[...] + y_ref[...]) + b_ref[...])
    h_ref[...] = h

    rid = i * BLK + lax.broadcasted_iota(jnp.int32, (BLK, 1), 0)
    hm = jnp.where(rid < NN, h, 0.0)

    @pl.when(i == 0)
    def _():
        acc_ref[...] = jnp.zeros_like(acc_ref)

    acc_ref[0:1, :] += jnp.sum(hm, axis=0, keepdims=True)
    acc_ref[1:2, :] += jnp.sum(hm * hm, axis=0, keepdims=True)

    @pl.when(i == NBLK - 1)
    def _():
        mean = acc_ref[0:1, :] / NN
        var = acc_ref[1:2, :] / NN - mean * mean
        scale = g_ref[...] / jnp.sqrt(var + 1e-5)
        shift = be_ref[...] - mean * scale
        ss_ref[...] = jnp.concatenate(
            [scale, shift, jnp.zeros((6, DD), jnp.float32)], axis=0)


def _p7(a0, a1, y, d0, d1, b, gamma, beta):
    return pl.pallas_call(
        _p7_body,
        grid=(NBLK,),
        in_specs=[
            pl.BlockSpec((BLK, DD), lambda i: (i, 0)),
            pl.BlockSpec((BLK, DD), lambda i: (i, 0)),
            pl.BlockSpec((BLK, DD), lambda i: (i, 0)),
            pl.BlockSpec((BLK, 1), lambda i: (i, 0)),
            pl.BlockSpec((BLK, 1), lambda i: (i, 0)),
            pl.BlockSpec((1, DD), lambda i: (0, 0)),
            pl.BlockSpec((1, DD), lambda i: (0, 0)),
            pl.BlockSpec((1, DD), lambda i: (0, 0)),
        ],
        out_specs=[
            pl.BlockSpec((BLK, DD), lambda i: (i, 0)),
            pl.BlockSpec((8, DD), lambda i: (0, 0)),
        ],
        out_shape=[
            jax.ShapeDtypeStruct((NP, DD), jnp.float32),
            jax.ShapeDtypeStruct((8, DD), jnp.float32),
        ],
        scratch_shapes=[pltpu.VMEM((8, DD), jnp.float32)],
    )(a0, a1, y, d0, d1, b, gamma, beta)


def _p8_body(bat_smem, h_ref, bat_ref, ss_ref, w1_ref, b1_ref, w2_ref,
             b2_ref, w3_ref, b3_ref, out_ref, gmax_ref):
    i = pl.program_id(0)

    @pl.when(i == 0)
    def _():
        gmax_ref[...] = jnp.full((NG, DD), -jnp.inf, jnp.float32)

    hn = h_ref[...] * ss_ref[0:1, :] + ss_ref[1:2, :]
    bat = bat_ref[...]

    glo = bat_smem[i * BLK]
    ghi = jnp.minimum(bat_smem[i * BLK + BLK - 1], NG - 1)

    def body(g, _):
        mask = bat == g
        mm = jnp.max(jnp.where(mask, hn, -jnp.inf), axis=0, keepdims=True)
        cur = gmax_ref[pl.ds(g, 1), :]
        gmax_ref[pl.ds(g, 1), :] = jnp.maximum(cur, mm)
        return 0

    lax.fori_loop(glo, ghi + 1, body, 0)

    @pl.when(i == NBLK - 1)
    def _():
        g = gmax_ref[...]
        g = jnp.where(g == -jnp.inf, 0.0, g)
        g = jax.nn.relu(_dot(g, w1_ref[...]) + b1_ref[...])
        g = jax.nn.relu(_dot(g, w2_ref[...]) + b2_ref[...])
        out_ref[...] = _dot(g, w3_ref[...]) + b3_ref[...]


def _p8(h3, batp, ss, lW1, lb1, lW2p, lb2p, lW3p, lb3p):
    full = lambda i, s: (0, 0)
    out, _ = pl.pallas_call(
        _p8_body,
        grid_spec=pltpu.PrefetchScalarGridSpec(
            num_scalar_prefetch=1,
            grid=(NBLK,),
            in_specs=[
                pl.BlockSpec((BLK, DD), lambda i, s: (i, 0)),
                pl.BlockSpec((BLK, 1), lambda i, s: (i, 0)),
                pl.BlockSpec((8, DD), full),
                pl.BlockSpec((DD, DD), full),
                pl.BlockSpec((1, DD), full),
                pl.BlockSpec((DD, DD), full),
                pl.BlockSpec((1, DD), full),
                pl.BlockSpec((DD, DD), full),
                pl.BlockSpec((1, DD), full),
            ],
            out_specs=[
                pl.BlockSpec((NG, DD), full),
                pl.BlockSpec((NG, DD), full),
            ],
        ),
        out_shape=[
            jax.ShapeDtypeStruct((NG, DD), jnp.float32),
            jax.ShapeDtypeStruct((NG, DD), jnp.float32),
        ],
    )(batp.reshape(NP), h3, batp, ss, lW1, lb1, lW2p, lb2p, lW3p, lb3p)
    return out


def kernel(x, edge_index, batch, W1, b1, W2, b2, W3, b3, gamma, beta,
           lW1, lb1, lW2, lb2, lW3, lb3):
    src = edge_index[0]
    dst = edge_index[1]
    npad = EPAD - EE
    sidx2 = jnp.concatenate([src, jnp.zeros((npad,), jnp.int32)]).reshape(
        NS, CPS, KCH)
    didx2 = jnp.concatenate([dst, jnp.full((npad,), NP - 1, jnp.int32)]).reshape(
        NS, CPS, KCH)
    eidx2 = jnp.concatenate([sidx2, didx2], axis=1)
    sw = jnp.zeros((NS, NC, CAM, KCH), jnp.int32)
    dw = jnp.full((NS, NC, CAM, KCH), NP - 1, jnp.int32)
    sw = sw.at[:, 0, :CA].set(sidx2[:, :CA]).at[:, 1, :CB].set(sidx2[:, CA:])
    dw = dw.at[:, 0, :CA].set(didx2[:, :CA]).at[:, 1, :CB].set(didx2[:, CA:])
    eidxw = jnp.concatenate([sw, dw], axis=2).reshape(NW, 2 * CAM, KCH)
    x_pad = jnp.concatenate([x, jnp.zeros((NP - NN, DD), jnp.float32)], axis=0)
    batp = jnp.concatenate([batch, jnp.full((NP - NN,), NG, jnp.int32)]).reshape(
        NP, 1)

    b1r = b1.reshape(1, DD)
    b2r = b2.reshape(1, DD)
    b3r = b3.reshape(1, DD)
    gr = gamma.reshape(1, DD)
    ber = beta.reshape(1, DD)
    lb1r = lb1.reshape(1, DD)
    lW2p = jnp.pad(lW2, ((0, 0), (0, DD - lW2.shape[1])))
    lb2p = jnp.pad(lb2, (0, DD - lb2.shape[0])).reshape(1, DD)
    lW3p = jnp.pad(lW3, ((0, DD - lW3.shape[0]), (0, DD - lW3.shape[1])))
    lb3p = jnp.pad(lb3, (0, DD - lb3.shape[0])).reshape(1, DD)

    degp = _deg_sc(eidx2)
    d0 = lax.slice(degp[0], (0, 0), (NP, 1))
    d1 = lax.slice(degp[1], (0, 0), (NP, 1))

    xw1 = _p1a(x_pad, W1)
    y1 = _p1b(xw1, d0, d1)
    a1 = _edge_sc(y1, eidxw)
    y2 = _p3(a1[0], a1[1], y1, d0, d1, b1r, W2)
    a2 = _edge_sc(y2, eidxw)
    y3 = _p3(a2[0], a2[1], y2, d0, d1, b2r, W3)
    a3 = _edge_sc(y3, eidxw)
    h3, ss = _p7(a3[0], a3[1], y3, d0, d1, b3r, gr, ber)
    out = _p8(h3, batp, ss, lW1, lb1r, lW2p, lb2p, lW3p, lb3p)
    return out[:, :NCLS]
